# baseline (speedup 1.0000x reference)
import jax
import jax.numpy as jnp
from jax import lax
from jax.experimental import pallas as pl
from jax.experimental.pallas import tpu as pltpu
from jax.experimental.pallas import tpu_sc as plsc

DIM = 8192
ROWS = 4096
NC, NS, L = 2, 16, 16
NW = NC * NS
RPW = ROWS // NW
BW = 128
SW = 64
MROWS = 8
NRB = RPW // MROWS
WMAX = 6144
MID_COLS = ((BW, 6144), (BW + 6144, 1792))
CHUNKS = tuple((rb * MROWS, c0, w) for rb in range(NRB) for (c0, w) in MID_COLS)
NMID = len(CHUNKS)
NBUF = 2


def _body(x, o, pmid, lb, rb, *sems):
    m_in = sems[:NBUF]
    m_out = sems[NBUF:2 * NBUF]
    s_lbg, s_rbg, s_lbs, s_rbs = sems[2 * NBUF:]
    wid = lax.axis_index("s") * NC + lax.axis_index("c")
    sid = lax.axis_index("s")
    base = wid * RPW

    glb = pltpu.make_async_copy(x.at[pl.ds(base, RPW), pl.ds(0, BW)], lb, s_lbg)
    grb = pltpu.make_async_copy(
        x.at[pl.ds(base, RPW), pl.ds(DIM - BW, BW)], rb, s_rbg)
    glb.start()
    grb.start()

    def mid_slice(ref, g):
        r0, c0, w = CHUNKS[g]
        return ref.at[pl.ds(base + r0, MROWS), pl.ds(c0, w)]

    def buf_slice(b, g):
        w = CHUNKS[g][2]
        return pmid.at[sid, b, pl.ds(0, MROWS), pl.ds(0, w)]

    def gmid(g, b):
        return pltpu.make_async_copy(mid_slice(x, g), buf_slice(b, g), m_in[b])

    def smid(g, b):
        return pltpu.make_async_copy(buf_slice(b, g), mid_slice(o, g), m_out[b])

    for b in range(NBUF):
        gmid(b, b).start()

    glb.wait()
    grb.wait()

    def row(r, carry):
        for v in range(SW // L):
            a = lb[r, pl.ds(L * v, L)]
            b_ = rb[r, pl.ds(BW - L * (v + 1), L)]
            lb[r, pl.ds(L * v, L)] = lax.rev(b_, (0,))
            rb[r, pl.ds(BW - L * (v + 1), L)] = lax.rev(a, (0,))
        return carry

    lax.fori_loop(0, RPW, row, 0)

    pltpu.make_async_copy(lb, o.at[pl.ds(base, RPW), pl.ds(0, BW)], s_lbs).start()
    pltpu.make_async_copy(
        rb, o.at[pl.ds(base, RPW), pl.ds(DIM - BW, BW)], s_rbs).start()

    for g in range(NMID):
        b = g % NBUF
        gmid(g, b).wait()
        smid(g, b).start()
        nxt = g + 1
        if NBUF <= nxt < NMID:
            nb = nxt % NBUF
            smid(nxt - NBUF, nb).wait()
            gmid(nxt, nb).start()

    for b in range(NBUF):
        smid(NMID - NBUF + b, b).wait()
    pltpu.make_async_copy(lb, o.at[pl.ds(base, RPW), pl.ds(0, BW)], s_lbs).wait()
    pltpu.make_async_copy(
        rb, o.at[pl.ds(base, RPW), pl.ds(DIM - BW, BW)], s_rbs).wait()


def kernel(x0, x1, x2):
    mesh = plsc.VectorSubcoreMesh(
        core_axis_name="c", subcore_axis_name="s",
        num_cores=NC, num_subcores=NS)
    k = pl.kernel(
        _body,
        out_type=jax.ShapeDtypeStruct((ROWS, DIM), jnp.float32),
        mesh=mesh,
        scratch_types=(
            [pltpu.VMEM_SHARED((NS, NBUF, MROWS, WMAX), jnp.float32),
             pltpu.VMEM((RPW, BW), jnp.float32),
             pltpu.VMEM((RPW, BW), jnp.float32)]
            + [pltpu.SemaphoreType.DMA] * (2 * NBUF + 4)
        ),
    )
    mixed = k(x0)
    return (mixed, x1, x2)


# speedup vs baseline: 1.0176x; 1.0176x over previous
import jax
import jax.numpy as jnp
from jax import lax
from jax.experimental import pallas as pl
from jax.experimental.pallas import tpu as pltpu
from jax.experimental.pallas import tpu_sc as plsc

DIM = 8192
ROWS = 4096
NC, NS, L = 2, 16, 16
NW = NC * NS
RPW = ROWS // NW
BW = 128
SW = 64
MROWS = 16
NRB = RPW // MROWS
WMAX = 2688
MID_COLS = ((BW, 2688), (BW + 2688, 2688), (BW + 5376, 2560))
CHUNKS = tuple((rb * MROWS, c0, w) for rb in range(NRB) for (c0, w) in MID_COLS)
NMID = len(CHUNKS)
NBUF = 2


def _body(x, o, pmid, lb, rb, *sems):
    m_in = sems[:NBUF]
    m_out = sems[NBUF:2 * NBUF]
    s_lbg, s_rbg, s_lbs, s_rbs = sems[2 * NBUF:]
    wid = lax.axis_index("s") * NC + lax.axis_index("c")
    sid = lax.axis_index("s")
    base = wid * RPW

    glb = pltpu.make_async_copy(x.at[pl.ds(base, RPW), pl.ds(0, BW)], lb, s_lbg)
    grb = pltpu.make_async_copy(
        x.at[pl.ds(base, RPW), pl.ds(DIM - BW, BW)], rb, s_rbg)
    glb.start()
    grb.start()

    def mid_slice(ref, g):
        r0, c0, w = CHUNKS[g]
        return ref.at[pl.ds(base + r0, MROWS), pl.ds(c0, w)]

    def buf_slice(b, g):
        w = CHUNKS[g][2]
        return pmid.at[sid, b, pl.ds(0, MROWS), pl.ds(0, w)]

    def gmid(g, b):
        return pltpu.make_async_copy(mid_slice(x, g), buf_slice(b, g), m_in[b])

    def smid(g, b):
        return pltpu.make_async_copy(buf_slice(b, g), mid_slice(o, g), m_out[b])

    for b in range(NBUF):
        gmid(b, b).start()

    glb.wait()
    grb.wait()

    def row(r, carry):
        for v in range(SW // L):
            a = lb[r, pl.ds(L * v, L)]
            b_ = rb[r, pl.ds(BW - L * (v + 1), L)]
            lb[r, pl.ds(L * v, L)] = lax.rev(b_, (0,))
            rb[r, pl.ds(BW - L * (v + 1), L)] = lax.rev(a, (0,))
        return carry

    lax.fori_loop(0, RPW, row, 0)

    pltpu.make_async_copy(lb, o.at[pl.ds(base, RPW), pl.ds(0, BW)], s_lbs).start()
    pltpu.make_async_copy(
        rb, o.at[pl.ds(base, RPW), pl.ds(DIM - BW, BW)], s_rbs).start()

    for g in range(NMID):
        b = g % NBUF
        gmid(g, b).wait()
        smid(g, b).start()
        nxt = g + 1
        if NBUF <= nxt < NMID:
            nb = nxt % NBUF
            smid(nxt - NBUF, nb).wait()
            gmid(nxt, nb).start()

    for b in range(NBUF):
        smid(NMID - NBUF + b, b).wait()
    pltpu.make_async_copy(lb, o.at[pl.ds(base, RPW), pl.ds(0, BW)], s_lbs).wait()
    pltpu.make_async_copy(
        rb, o.at[pl.ds(base, RPW), pl.ds(DIM - BW, BW)], s_rbs).wait()


def kernel(x0, x1, x2):
    mesh = plsc.VectorSubcoreMesh(
        core_axis_name="c", subcore_axis_name="s",
        num_cores=NC, num_subcores=NS)
    k = pl.kernel(
        _body,
        out_type=jax.ShapeDtypeStruct((ROWS, DIM), jnp.float32),
        mesh=mesh,
        scratch_types=(
            [pltpu.VMEM_SHARED((NS, NBUF, MROWS, WMAX), jnp.float32),
             pltpu.VMEM((RPW, BW), jnp.float32),
             pltpu.VMEM((RPW, BW), jnp.float32)]
            + [pltpu.SemaphoreType.DMA] * (2 * NBUF + 4)
        ),
    )
    mixed = k(x0)
    return (mixed, x1, x2)
